# baseline (speedup 1.0000x reference)
import jax
import jax.numpy as jnp
from jax import lax
from jax.experimental import pallas as pl
from jax.experimental.pallas import tpu as pltpu
from jax.experimental.pallas import tpu_sc as plsc

B = 4096
D = 128
N = 16
NR = 32
NC = 2
NS = 16
NW = NC * NS
CHUNK = 128
BPW = B // NW
L = 16


def _mesh():
    return plsc.VectorSubcoreMesh(core_axis_name="c", subcore_axis_name="s")


def _sc_body(user_idx, item_idx, adjcat, user_emb, ent_emb,
             u_out, ev0_out, ev1_out, ev2_out, r1_out, r2_out,
             idx_v, e1f_v, e2f_v, hi_v, lo_v, dstbuf, rows_v, rbuf, sem):
    wid = lax.axis_index("s") * NC + lax.axis_index("c")
    base = wid * BPW
    iota = lax.iota(jnp.int32, L)

    def expand_chunk(load_ids, scatter_e):
        for g in range(CHUNK // L):
            v = load_ids(g * L + iota)
            plsc.store_scatter(hi_v, [g * L + iota], v >> 2)
            plsc.store_scatter(lo_v, [g * L + iota], (v & 3) << 5)
        pltpu.async_copy(adjcat.at[hi_v], dstbuf, sem).wait()
        for g in range(CHUNK // L):
            rows = g * L + iota
            lo = plsc.load_gather(lo_v, [rows])
            for j in range(N):
                e_j = plsc.load_gather(dstbuf, [rows, lo + j])
                r_j = plsc.load_gather(dstbuf, [rows, lo + N + j])
                scatter_e(rows, j, e_j)
                plsc.store_scatter(
                    rbuf, [rows, jnp.full((L,), j, jnp.int32)], r_j)

    pltpu.sync_copy(item_idx.at[pl.ds(base, BPW)], idx_v)
    expand_chunk(
        lambda off: plsc.load_gather(idx_v, [off]),
        lambda rows, j, e_j: plsc.store_scatter(e1f_v, [rows * N + j], e_j))
    pltpu.sync_copy(rbuf, r1_out.at[pl.ds(base, BPW)])

    pltpu.async_copy(ent_emb.at[idx_v], rows_v, sem).wait()
    pltpu.sync_copy(rows_v, ev0_out.at[pl.ds(base, BPW)])
    pltpu.sync_copy(user_idx.at[pl.ds(base, BPW)], idx_v)
    pltpu.async_copy(user_emb.at[idx_v], rows_v, sem).wait()
    pltpu.sync_copy(rows_v, u_out.at[pl.ds(base, BPW)])

    def ev1_body(c, carry):
        pltpu.async_copy(ent_emb.at[e1f_v.at[pl.ds(c * CHUNK, CHUNK)]],
                         rows_v, sem).wait()
        pltpu.sync_copy(rows_v,
                        ev1_out.at[pl.ds(base * N + c * CHUNK, CHUNK)])
        return carry

    lax.fori_loop(0, (BPW * N) // CHUNK, ev1_body, 0)

    def exp2_body(c, carry):
        expand_chunk(
            lambda off: plsc.load_gather(e1f_v, [c * CHUNK + off]),
            lambda rows, j, e_j: plsc.store_scatter(
                e2f_v, [(c * CHUNK + rows) * N + j], e_j))
        pltpu.sync_copy(rbuf, r2_out.at[pl.ds(base * N + c * CHUNK, CHUNK)])
        return carry

    lax.fori_loop(0, (BPW * N) // CHUNK, exp2_body, 0)

    def ev2_nn_body(nn, carry):
        def ev2_c_body(c, carry2):
            for g in range(CHUNK // L):
                qs = c * CHUNK + g * L + iota
                ids = plsc.load_gather(e2f_v, [qs * N + nn])
                plsc.store_scatter(hi_v, [g * L + iota], ids)
            pltpu.async_copy(ent_emb.at[hi_v], rows_v, sem).wait()
            pltpu.sync_copy(
                rows_v,
                ev2_out.at[nn, pl.ds(base * N + c * CHUNK, CHUNK)])
            return carry2

        lax.fori_loop(0, (BPW * N) // CHUNK, ev2_c_body, 0)
        return carry

    lax.fori_loop(0, N, ev2_nn_body, 0)


def _sc_gathers(user_idx, item_idx, adjcat, user_emb, ent_emb):
    return pl.kernel(
        _sc_body,
        out_type=[
            jax.ShapeDtypeStruct((B, D), jnp.float32),
            jax.ShapeDtypeStruct((B, D), jnp.float32),
            jax.ShapeDtypeStruct((B * N, D), jnp.float32),
            jax.ShapeDtypeStruct((N, B * N, D), jnp.float32),
            jax.ShapeDtypeStruct((B, N), jnp.int32),
            jax.ShapeDtypeStruct((B * N, N), jnp.int32),
        ],
        mesh=_mesh(),
        compiler_params=pltpu.CompilerParams(needs_layout_passes=False),
        scratch_types=[
            pltpu.VMEM((BPW,), jnp.int32),
            pltpu.VMEM((BPW * N,), jnp.int32),
            pltpu.VMEM((BPW * N * N,), jnp.int32),
            pltpu.VMEM((CHUNK,), jnp.int32),
            pltpu.VMEM((CHUNK,), jnp.int32),
            pltpu.VMEM((CHUNK, D), jnp.int32),
            pltpu.VMEM((CHUNK, D), jnp.float32),
            pltpu.VMEM((CHUNK, N), jnp.int32),
            pltpu.SemaphoreType.DMA,
        ],
    )(user_idx, item_idx, adjcat, user_emb, ent_emb)


BB = 256


def _softmax(x):
    m = jnp.max(x, axis=-1, keepdims=True)
    e = jnp.exp(x - m)
    return e / jnp.sum(e, axis=-1, keepdims=True)


def _tc_body(u_ref, ev0_ref, ev1_ref, ev2_ref, r1_ref, r2_ref,
             rel_ref, w0_ref, b0_ref, w1_ref, b1_ref, out_ref,
             s0_scr, s1_scr, agg1_scr):
    f32 = jnp.float32
    nn = pl.program_id(1)

    @pl.when(nn == 0)
    def _scores():
        u = u_ref[...]
        rel = rel_ref[...]
        P = lax.dot_general(u, rel, (((1,), (1,)), ((), ())),
                            preferred_element_type=f32)
        iota_r = lax.broadcasted_iota(jnp.int32, (1, 1, NR), 2)
        r1 = r1_ref[...]
        oh1 = (r1[:, :, None] == iota_r).astype(f32)
        s0_scr[...] = _softmax(jnp.sum(oh1 * P[:, None, :], axis=-1))
        r2 = r2_ref[...]
        Pexp = jnp.broadcast_to(P[:, None, :],
                                (BB, N, NR)).reshape(BB * N, NR)
        oh2 = (r2[:, :, None] == iota_r).astype(f32)
        s1_scr[...] = _softmax(jnp.sum(oh2 * Pexp[:, None, :], axis=-1))

    nn_mask = (lax.broadcasted_iota(jnp.int32, (1, N), 1) == nn).astype(f32)
    w_nn = jnp.sum(s1_scr[...] * nn_mask, axis=1, keepdims=True)
    contrib = ev2_ref[0] * w_nn

    @pl.when(nn == 0)
    def _init():
        agg1_scr[...] = contrib

    @pl.when(nn > 0)
    def _acc():
        agg1_scr[...] = agg1_scr[...] + contrib

    @pl.when(nn == N - 1)
    def _tail():
        u = u_ref[...]
        s0 = s0_scr[...]
        ev1 = ev1_ref[...].reshape(BB * N, D)
        w0 = w0_ref[...]
        b0 = b0_ref[...]
        h1 = jax.nn.relu(jnp.dot(ev1 + agg1_scr[...], w0,
                                 preferred_element_type=f32) + b0)
        agg0 = jnp.sum(ev1.reshape(BB, N, D) * s0[:, :, None], axis=1)
        h0 = jax.nn.relu(jnp.dot(ev0_ref[...] + agg0, w0,
                                 preferred_element_type=f32) + b0)
        agg0b = jnp.sum(h1.reshape(BB, N, D) * s0[:, :, None], axis=1)
        outv = jnp.tanh(jnp.dot(h0 + agg0b, w1_ref[...],
                                preferred_element_type=f32) + b1_ref[...])
        logits = jnp.sum(u * outv, axis=-1)
        out_ref[...] = jax.nn.sigmoid(logits)[None, None, :]


def _tc_dense(u, ev0, ev1_3, ev2_2, r1, r2, rel, W0, b0, W1, b1):
    nb = B // BB
    const = lambda *_: (0, 0)
    return pl.pallas_call(
        _tc_body,
        grid=(nb, N),
        in_specs=[
            pl.BlockSpec((BB, D), lambda i, nn: (i, 0)),
            pl.BlockSpec((BB, D), lambda i, nn: (i, 0)),
            pl.BlockSpec((BB, N, D), lambda i, nn: (i, 0, 0)),
            pl.BlockSpec((1, BB * N, D), lambda i, nn: (nn, i, 0)),
            pl.BlockSpec((BB, N), lambda i, nn: (i, 0)),
            pl.BlockSpec((BB * N, N), lambda i, nn: (i, 0)),
            pl.BlockSpec((NR, D), lambda i, nn: (0, 0)),
            pl.BlockSpec((D, D), lambda i, nn: (0, 0)),
            pl.BlockSpec((1, D), lambda i, nn: (0, 0)),
            pl.BlockSpec((D, D), lambda i, nn: (0, 0)),
            pl.BlockSpec((1, D), lambda i, nn: (0, 0)),
        ],
        out_specs=pl.BlockSpec((1, 1, BB), lambda i, nn: (i, 0, 0)),
        out_shape=jax.ShapeDtypeStruct((nb, 1, BB), jnp.float32),
        scratch_shapes=[
            pltpu.VMEM((BB, N), jnp.float32),
            pltpu.VMEM((BB * N, N), jnp.float32),
            pltpu.VMEM((BB * N, D), jnp.float32),
        ],
    )(u, ev0, ev1_3, ev2_2, r1, r2, rel, W0, b0, W1, b1)


def kernel(user_indices, item_indices, adj_entity, adj_relation,
           user_emb, entity_emb, relation_emb, W0, b0, W1, b1):
    adjcat = jnp.concatenate([adj_entity, adj_relation], axis=1)
    adjcat = adjcat.reshape(adj_entity.shape[0] // 4, 128)
    u, ev0, ev1, ev2, r1, r2 = _sc_gathers(
        user_indices, item_indices, adjcat, user_emb, entity_emb)
    out = _tc_dense(u, ev0, ev1.reshape(B, N, D), ev2, r1, r2, relation_emb,
                    W0, b0.reshape(1, D), W1, b1.reshape(1, D))
    return out.reshape(B)


# speedup vs baseline: 1.0838x; 1.0838x over previous
import jax
import jax.numpy as jnp
from jax import lax
from jax.experimental import pallas as pl
from jax.experimental.pallas import tpu as pltpu
from jax.experimental.pallas import tpu_sc as plsc

B = 4096
D = 128
N = 16
NR = 32
NC = 2
NS = 16
NW = NC * NS
CHUNK = 128
BPW = B // NW
L = 16


def _mesh():
    return plsc.VectorSubcoreMesh(core_axis_name="c", subcore_axis_name="s")


def _sc_body(user_idx, item_idx, adjcat, user_emb, ent_emb,
             u_out, ev0_out, ev1_out, ev2_out, r1_out, r2_out,
             idx_v, e1f_v, e2f_v, hi_v, lo_v, dstbuf, rows_v, rbuf, sem):
    wid = lax.axis_index("s") * NC + lax.axis_index("c")
    base = wid * BPW
    iota = lax.iota(jnp.int32, L)

    def expand_chunk(load_ids, scatter_e):
        for g in range(CHUNK // L):
            v = load_ids(g * L + iota)
            plsc.store_scatter(hi_v, [g * L + iota], v >> 2)
            plsc.store_scatter(lo_v, [g * L + iota], (v & 3) << 5)
        pltpu.async_copy(adjcat.at[hi_v], dstbuf, sem).wait()
        for g in range(CHUNK // L):
            rows = g * L + iota
            lo = plsc.load_gather(lo_v, [rows])
            for j in range(N):
                e_j = plsc.load_gather(dstbuf, [rows, lo + j])
                r_j = plsc.load_gather(dstbuf, [rows, lo + N + j])
                scatter_e(rows, j, e_j)
                plsc.store_scatter(
                    rbuf, [rows, jnp.full((L,), j, jnp.int32)], r_j)

    pltpu.sync_copy(item_idx.at[pl.ds(base, BPW)], idx_v)
    expand_chunk(
        lambda off: plsc.load_gather(idx_v, [off]),
        lambda rows, j, e_j: plsc.store_scatter(e1f_v, [rows * N + j], e_j))
    pltpu.sync_copy(rbuf, r1_out.at[pl.ds(base, BPW)])

    pltpu.async_copy(ent_emb.at[idx_v], rows_v, sem).wait()
    pltpu.sync_copy(rows_v, ev0_out.at[pl.ds(base, BPW)])
    pltpu.sync_copy(user_idx.at[pl.ds(base, BPW)], idx_v)
    pltpu.async_copy(user_emb.at[idx_v], rows_v, sem).wait()
    pltpu.sync_copy(rows_v, u_out.at[pl.ds(base, BPW)])

    def ev1_body(c, carry):
        pltpu.async_copy(ent_emb.at[e1f_v.at[pl.ds(c * CHUNK, CHUNK)]],
                         rows_v, sem).wait()
        pltpu.sync_copy(rows_v,
                        ev1_out.at[pl.ds(base * N + c * CHUNK, CHUNK)])
        return carry

    lax.fori_loop(0, (BPW * N) // CHUNK, ev1_body, 0)

    def exp2_body(c, carry):
        expand_chunk(
            lambda off: plsc.load_gather(e1f_v, [c * CHUNK + off]),
            lambda rows, j, e_j: plsc.store_scatter(
                e2f_v, [(c * CHUNK + rows) * N + j], e_j))
        pltpu.sync_copy(rbuf, r2_out.at[pl.ds(base * N + c * CHUNK, CHUNK)])
        return carry

    lax.fori_loop(0, (BPW * N) // CHUNK, exp2_body, 0)

    def ev2_nn_body(nn, carry):
        def ev2_c_body(c, carry2):
            for g in range(CHUNK // L):
                qs = c * CHUNK + g * L + iota
                ids = plsc.load_gather(e2f_v, [qs * N + nn])
                plsc.store_scatter(hi_v, [g * L + iota], ids)
            pltpu.async_copy(ent_emb.at[hi_v], rows_v, sem).wait()
            pltpu.sync_copy(
                rows_v,
                ev2_out.at[nn, pl.ds(base * N + c * CHUNK, CHUNK)])
            return carry2

        lax.fori_loop(0, (BPW * N) // CHUNK, ev2_c_body, 0)
        return carry

    lax.fori_loop(0, N, ev2_nn_body, 0)


def _sc_gathers(user_idx, item_idx, adjcat, user_emb, ent_emb):
    return pl.kernel(
        _sc_body,
        out_type=[
            jax.ShapeDtypeStruct((B, D), jnp.float32),
            jax.ShapeDtypeStruct((B, D), jnp.float32),
            jax.ShapeDtypeStruct((B * N, D), jnp.float32),
            jax.ShapeDtypeStruct((N, B * N, D), jnp.float32),
            jax.ShapeDtypeStruct((B, N), jnp.int32),
            jax.ShapeDtypeStruct((B * N, N), jnp.int32),
        ],
        mesh=_mesh(),
        compiler_params=pltpu.CompilerParams(needs_layout_passes=False),
        scratch_types=[
            pltpu.VMEM((BPW,), jnp.int32),
            pltpu.VMEM((BPW * N,), jnp.int32),
            pltpu.VMEM((BPW * N * N,), jnp.int32),
            pltpu.VMEM((CHUNK,), jnp.int32),
            pltpu.VMEM((CHUNK,), jnp.int32),
            pltpu.VMEM((CHUNK, D), jnp.int32),
            pltpu.VMEM((CHUNK, D), jnp.float32),
            pltpu.VMEM((CHUNK, N), jnp.int32),
            pltpu.SemaphoreType.DMA,
        ],
    )(user_idx, item_idx, adjcat, user_emb, ent_emb)


BB = 64


def _softmax(x):
    m = jnp.max(x, axis=-1, keepdims=True)
    e = jnp.exp(x - m)
    return e / jnp.sum(e, axis=-1, keepdims=True)


def _tc_body(u_ref, ev0_ref, ev1_ref, ev2_ref, r1_ref, r2_ref,
             rel_ref, w0_ref, b0_ref, w1_ref, b1_ref, out_ref,
             s0_scr, s1_scr, agg1_scr):
    f32 = jnp.float32
    nn = pl.program_id(1)

    @pl.when(nn == 0)
    def _scores():
        u = u_ref[...]
        rel = rel_ref[...]
        P = lax.dot_general(u, rel, (((1,), (1,)), ((), ())),
                            preferred_element_type=f32)
        iota_r = lax.broadcasted_iota(jnp.int32, (1, 1, NR), 2)
        r1 = r1_ref[...]
        oh1 = (r1[:, :, None] == iota_r).astype(f32)
        s0_scr[...] = _softmax(jnp.sum(oh1 * P[:, None, :], axis=-1))
        r2 = r2_ref[...]
        Pexp = jnp.broadcast_to(P[:, None, :],
                                (BB, N, NR)).reshape(BB * N, NR)
        oh2 = (r2[:, :, None] == iota_r).astype(f32)
        s1_scr[...] = _softmax(jnp.sum(oh2 * Pexp[:, None, :], axis=-1))

    nn_mask = (lax.broadcasted_iota(jnp.int32, (1, N), 1) == nn).astype(f32)
    w_nn = jnp.sum(s1_scr[...] * nn_mask, axis=1, keepdims=True)
    contrib = ev2_ref[0] * w_nn

    @pl.when(nn == 0)
    def _init():
        agg1_scr[...] = contrib

    @pl.when(nn > 0)
    def _acc():
        agg1_scr[...] = agg1_scr[...] + contrib

    @pl.when(nn == N - 1)
    def _tail():
        u = u_ref[...]
        s0 = s0_scr[...]
        ev1 = ev1_ref[...].reshape(BB * N, D)
        w0 = w0_ref[...]
        b0 = b0_ref[...]
        h1 = jax.nn.relu(jnp.dot(ev1 + agg1_scr[...], w0,
                                 preferred_element_type=f32) + b0)
        agg0 = jnp.sum(ev1.reshape(BB, N, D) * s0[:, :, None], axis=1)
        h0 = jax.nn.relu(jnp.dot(ev0_ref[...] + agg0, w0,
                                 preferred_element_type=f32) + b0)
        agg0b = jnp.sum(h1.reshape(BB, N, D) * s0[:, :, None], axis=1)
        outv = jnp.tanh(jnp.dot(h0 + agg0b, w1_ref[...],
                                preferred_element_type=f32) + b1_ref[...])
        logits = jnp.sum(u * outv, axis=-1)
        out_ref[...] = jax.nn.sigmoid(logits)[None, None, :]


def _tc_dense(u, ev0, ev1_3, ev2_2, r1, r2, rel, W0, b0, W1, b1):
    nb = B // BB
    const = lambda *_: (0, 0)
    return pl.pallas_call(
        _tc_body,
        grid=(nb, N),
        in_specs=[
            pl.BlockSpec((BB, D), lambda i, nn: (i, 0)),
            pl.BlockSpec((BB, D), lambda i, nn: (i, 0)),
            pl.BlockSpec((BB, N, D), lambda i, nn: (i, 0, 0)),
            pl.BlockSpec((1, BB * N, D), lambda i, nn: (nn, i, 0)),
            pl.BlockSpec((BB, N), lambda i, nn: (i, 0)),
            pl.BlockSpec((BB * N, N), lambda i, nn: (i, 0)),
            pl.BlockSpec((NR, D), lambda i, nn: (0, 0)),
            pl.BlockSpec((D, D), lambda i, nn: (0, 0)),
            pl.BlockSpec((1, D), lambda i, nn: (0, 0)),
            pl.BlockSpec((D, D), lambda i, nn: (0, 0)),
            pl.BlockSpec((1, D), lambda i, nn: (0, 0)),
        ],
        out_specs=pl.BlockSpec((1, 1, BB), lambda i, nn: (i, 0, 0)),
        out_shape=jax.ShapeDtypeStruct((nb, 1, BB), jnp.float32),
        scratch_shapes=[
            pltpu.VMEM((BB, N), jnp.float32),
            pltpu.VMEM((BB * N, N), jnp.float32),
            pltpu.VMEM((BB * N, D), jnp.float32),
        ],
    )(u, ev0, ev1_3, ev2_2, r1, r2, rel, W0, b0, W1, b1)


def kernel(user_indices, item_indices, adj_entity, adj_relation,
           user_emb, entity_emb, relation_emb, W0, b0, W1, b1):
    adjcat = jnp.concatenate([adj_entity, adj_relation], axis=1)
    adjcat = adjcat.reshape(adj_entity.shape[0] // 4, 128)
    u, ev0, ev1, ev2, r1, r2 = _sc_gathers(
        user_indices, item_indices, adjcat, user_emb, entity_emb)
    out = _tc_dense(u, ev0, ev1.reshape(B, N, D), ev2, r1, r2, relation_emb,
                    W0, b0.reshape(1, D), W1, b1.reshape(1, D))
    return out.reshape(B)


# speedup vs baseline: 2.6722x; 2.4656x over previous
import jax
import jax.numpy as jnp
from jax import lax
from jax.experimental import pallas as pl
from jax.experimental.pallas import tpu as pltpu
from jax.experimental.pallas import tpu_sc as plsc

B = 4096
D = 128
N = 16
NR = 32
NC = 2
NS = 16
NW = NC * NS
CHUNK = 128
BPW = B // NW
QPW = BPW * N
L = 16
DC = D // L


def _mesh():
    return plsc.VectorSubcoreMesh(core_axis_name="c", subcore_axis_name="s")


def _wid():
    return lax.axis_index("s") * NC + lax.axis_index("c")


def _sca_body(user_idx, item_idx, adjcat, user_emb, ent_emb,
              u_out, ev0_out, ev1_out, r1_out, e2f_out, r2f_out,
              idx_v, e1f_v, e2f_v, r2f_v, hi_v, lo_v, dstbuf, rows_v, rbuf,
              sem):
    base = _wid() * BPW
    iota = lax.iota(jnp.int32, L)

    def expand_chunk(load_ids, scatter_e, scatter_r):
        for g in range(CHUNK // L):
            v = load_ids(g * L + iota)
            plsc.store_scatter(hi_v, [g * L + iota], v >> 2)
            plsc.store_scatter(lo_v, [g * L + iota], (v & 3) << 5)
        pltpu.async_copy(adjcat.at[hi_v], dstbuf, sem).wait()
        for g in range(CHUNK // L):
            rows = g * L + iota
            lo = plsc.load_gather(lo_v, [rows])
            for j in range(N):
                e_j = plsc.load_gather(dstbuf, [rows, lo + j])
                r_j = plsc.load_gather(dstbuf, [rows, lo + N + j])
                scatter_e(rows, j, e_j)
                scatter_r(rows, j, r_j)

    pltpu.sync_copy(item_idx.at[pl.ds(base, BPW)], idx_v)
    expand_chunk(
        lambda off: plsc.load_gather(idx_v, [off]),
        lambda rows, j, e_j: plsc.store_scatter(e1f_v, [rows * N + j], e_j),
        lambda rows, j, r_j: plsc.store_scatter(
            rbuf, [rows, jnp.full((L,), j, jnp.int32)], r_j))
    pltpu.sync_copy(rbuf, r1_out.at[pl.ds(base, BPW)])

    pltpu.async_copy(ent_emb.at[idx_v], rows_v, sem).wait()
    pltpu.sync_copy(rows_v, ev0_out.at[pl.ds(base, BPW)])
    pltpu.sync_copy(user_idx.at[pl.ds(base, BPW)], idx_v)
    pltpu.async_copy(user_emb.at[idx_v], rows_v, sem).wait()
    pltpu.sync_copy(rows_v, u_out.at[pl.ds(base, BPW)])

    def ev1_body(c, carry):
        pltpu.async_copy(ent_emb.at[e1f_v.at[pl.ds(c * CHUNK, CHUNK)]],
                         rows_v, sem).wait()
        pltpu.sync_copy(rows_v,
                        ev1_out.at[pl.ds(base * N + c * CHUNK, CHUNK)])
        return carry

    lax.fori_loop(0, QPW // CHUNK, ev1_body, 0)

    def exp2_body(c, carry):
        expand_chunk(
            lambda off: plsc.load_gather(e1f_v, [c * CHUNK + off]),
            lambda rows, j, e_j: plsc.store_scatter(
                e2f_v, [(c * CHUNK + rows) * N + j], e_j),
            lambda rows, j, r_j: plsc.store_scatter(
                r2f_v, [(c * CHUNK + rows) * N + j], r_j))
        return carry

    lax.fori_loop(0, QPW // CHUNK, exp2_body, 0)
    pltpu.sync_copy(e2f_v, e2f_out.at[pl.ds(base * N * N, QPW * N)])
    pltpu.sync_copy(r2f_v, r2f_out.at[pl.ds(base * N * N, QPW * N)])


def _sc_a(user_idx, item_idx, adjcat, user_emb, ent_emb):
    return pl.kernel(
        _sca_body,
        out_type=[
            jax.ShapeDtypeStruct((B, D), jnp.float32),
            jax.ShapeDtypeStruct((B, D), jnp.float32),
            jax.ShapeDtypeStruct((B * N, D), jnp.float32),
            jax.ShapeDtypeStruct((B, N), jnp.int32),
            jax.ShapeDtypeStruct((B * N * N,), jnp.int32),
            jax.ShapeDtypeStruct((B * N * N,), jnp.int32),
        ],
        mesh=_mesh(),
        compiler_params=pltpu.CompilerParams(needs_layout_passes=False),
        scratch_types=[
            pltpu.VMEM((BPW,), jnp.int32),
            pltpu.VMEM((QPW,), jnp.int32),
            pltpu.VMEM((QPW * N,), jnp.int32),
            pltpu.VMEM((QPW * N,), jnp.int32),
            pltpu.VMEM((CHUNK,), jnp.int32),
            pltpu.VMEM((CHUNK,), jnp.int32),
            pltpu.VMEM((CHUNK, D), jnp.int32),
            pltpu.VMEM((CHUNK, D), jnp.float32),
            pltpu.VMEM((BPW, N), jnp.int32),
            pltpu.SemaphoreType.DMA,
        ],
    )(user_idx, item_idx, adjcat, user_emb, ent_emb)


def _scb_body(e2f, r2f, p_hbm, ent_emb, agg1_out,
              idx_v, r2f_v, p_v, rows_v, aggbuf, sem):
    wid = _wid()
    iota = lax.iota(jnp.int32, L)
    pltpu.sync_copy(e2f.at[pl.ds(wid * QPW * N, QPW * N)], idx_v)
    pltpu.sync_copy(r2f.at[pl.ds(wid * QPW * N, QPW * N)], r2f_v)
    pltpu.sync_copy(p_hbm.at[pl.ds(wid * BPW, BPW)], p_v)

    def chunk_body(c, carry):
        pltpu.async_copy(ent_emb.at[idx_v.at[pl.ds(c * CHUNK, CHUNK)]],
                         rows_v, sem).wait()

        def q_body(qq, carry2):
            q = c * (CHUNK // N) + qq
            r2vec = plsc.load_gather(r2f_v, [q * N + iota])
            raw = plsc.load_gather(
                p_v, [jnp.full((L,), q >> 4, jnp.int32), r2vec])
            m = jnp.max(raw)
            ex = jnp.exp(raw - m)
            s = ex / jnp.sum(ex)
            accs = [jnp.zeros((L,), jnp.float32) for _ in range(DC)]
            for nn in range(N):
                w_nn = jnp.broadcast_to(s[nn], (L,))
                row = jnp.full((L,), qq * N + nn, jnp.int32)
                for dc in range(DC):
                    val = plsc.load_gather(rows_v, [row, dc * L + iota])
                    accs[dc] = accs[dc] + w_nn * val
            for dc in range(DC):
                plsc.store_scatter(
                    aggbuf, [jnp.full((L,), qq, jnp.int32), dc * L + iota],
                    accs[dc])
            return carry2

        lax.fori_loop(0, CHUNK // N, q_body, 0)
        pltpu.sync_copy(
            aggbuf,
            agg1_out.at[pl.ds(wid * QPW + c * (CHUNK // N), CHUNK // N)])
        return carry

    lax.fori_loop(0, (QPW * N) // CHUNK, chunk_body, 0)


def _sc_b(e2f, r2f, P, ent_emb):
    return pl.kernel(
        _scb_body,
        out_type=jax.ShapeDtypeStruct((B * N, D), jnp.float32),
        mesh=_mesh(),
        compiler_params=pltpu.CompilerParams(needs_layout_passes=False),
        scratch_types=[
            pltpu.VMEM((QPW * N,), jnp.int32),
            pltpu.VMEM((QPW * N,), jnp.int32),
            pltpu.VMEM((BPW, NR), jnp.float32),
            pltpu.VMEM((CHUNK, D), jnp.float32),
            pltpu.VMEM((CHUNK // N, D), jnp.float32),
            pltpu.SemaphoreType.DMA,
        ],
    )(e2f, r2f, P, ent_emb)


def _tcp_body(u_ref, rel_ref, p_ref):
    p_ref[...] = lax.dot_general(u_ref[...], rel_ref[...],
                                 (((1,), (1,)), ((), ())),
                                 preferred_element_type=jnp.float32)


def _tc_p(u, rel):
    return pl.pallas_call(
        _tcp_body,
        grid=(1,),
        in_specs=[pl.BlockSpec((B, D), lambda i: (0, 0)),
                  pl.BlockSpec((NR, D), lambda i: (0, 0))],
        out_specs=pl.BlockSpec((B, NR), lambda i: (0, 0)),
        out_shape=jax.ShapeDtypeStruct((B, NR), jnp.float32),
    )(u, rel)


BB = 128


def _softmax(x):
    m = jnp.max(x, axis=-1, keepdims=True)
    e = jnp.exp(x - m)
    return e / jnp.sum(e, axis=-1, keepdims=True)


def _tcf_body(u_ref, ev0_ref, ev1_ref, agg1_ref, r1_ref, p_ref,
              w0_ref, b0_ref, w1_ref, b1_ref, out_ref):
    f32 = jnp.float32
    u = u_ref[...]
    P = p_ref[...]
    iota_r = lax.broadcasted_iota(jnp.int32, (1, 1, NR), 2)
    r1 = r1_ref[...]
    oh1 = (r1[:, :, None] == iota_r).astype(f32)
    s0 = _softmax(jnp.sum(oh1 * P[:, None, :], axis=-1))

    ev1 = ev1_ref[...].reshape(BB * N, D)
    w0 = w0_ref[...]
    b0 = b0_ref[...]
    h1 = jax.nn.relu(jnp.dot(ev1 + agg1_ref[...], w0,
                             preferred_element_type=f32) + b0)
    agg0 = jnp.sum(ev1.reshape(BB, N, D) * s0[:, :, None], axis=1)
    h0 = jax.nn.relu(jnp.dot(ev0_ref[...] + agg0, w0,
                             preferred_element_type=f32) + b0)
    agg0b = jnp.sum(h1.reshape(BB, N, D) * s0[:, :, None], axis=1)
    outv = jnp.tanh(jnp.dot(h0 + agg0b, w1_ref[...],
                            preferred_element_type=f32) + b1_ref[...])
    logits = jnp.sum(u * outv, axis=-1)
    out_ref[...] = jax.nn.sigmoid(logits)[None, None, :]


def _tc_final(u, ev0, ev1_3, agg1, r1, P, W0, b0, W1, b1):
    nb = B // BB
    return pl.pallas_call(
        _tcf_body,
        grid=(nb,),
        in_specs=[
            pl.BlockSpec((BB, D), lambda i: (i, 0)),
            pl.BlockSpec((BB, D), lambda i: (i, 0)),
            pl.BlockSpec((BB, N, D), lambda i: (i, 0, 0)),
            pl.BlockSpec((BB * N, D), lambda i: (i, 0)),
            pl.BlockSpec((BB, N), lambda i: (i, 0)),
            pl.BlockSpec((BB, NR), lambda i: (i, 0)),
            pl.BlockSpec((D, D), lambda i: (0, 0)),
            pl.BlockSpec((1, D), lambda i: (0, 0)),
            pl.BlockSpec((D, D), lambda i: (0, 0)),
            pl.BlockSpec((1, D), lambda i: (0, 0)),
        ],
        out_specs=pl.BlockSpec((1, 1, BB), lambda i: (i, 0, 0)),
        out_shape=jax.ShapeDtypeStruct((nb, 1, BB), jnp.float32),
    )(u, ev0, ev1_3, agg1, r1, P, W0, b0, W1, b1)


def kernel(user_indices, item_indices, adj_entity, adj_relation,
           user_emb, entity_emb, relation_emb, W0, b0, W1, b1):
    adjcat = jnp.concatenate([adj_entity, adj_relation], axis=1)
    adjcat = adjcat.reshape(adj_entity.shape[0] // 4, 128)
    u, ev0, ev1, r1, e2f, r2f = _sc_a(
        user_indices, item_indices, adjcat, user_emb, entity_emb)
    P = _tc_p(u, relation_emb)
    agg1 = _sc_b(e2f, r2f, P, entity_emb)
    out = _tc_final(u, ev0, ev1.reshape(B, N, D), agg1, r1, P,
                    W0, b0.reshape(1, D), W1, b1.reshape(1, D))
    return out.reshape(B)
